# Initial kernel scaffold; baseline (speedup 1.0000x reference)
import jax
import jax.numpy as jnp
from jax.experimental import pallas as pl


def kernel(edge_index, h_src, h_dst, rel_weight, W1, b1, W2, b2):
    raise NotImplementedError("write your pallas kernel here")


# speedup vs baseline: 3.4473x; 3.4473x over previous
import functools

import jax
import jax.numpy as jnp
from jax import lax
from jax.experimental import pallas as pl
from jax.experimental.pallas import tpu as pltpu
from jax.experimental.pallas import tpu_sc as plsc

FEAT = 32
HID = 128

NC = 2
NS = 16
NW = NC * NS

CB = 80


def _sc_gather_body(nchunk, per_w, src_hbm, dst_hbm, hsrc_hbm, hdst_hbm,
                    xs_hbm, xd_hbm, idx_s, idx_d, rows_s, rows_d, sem_s, sem_d):
    wid = lax.axis_index("s") * NC + lax.axis_index("c")

    def body(c, carry):
        base = wid * per_w + c * CB
        pltpu.sync_copy(src_hbm.at[pl.ds(base, CB)], idx_s)
        pltpu.sync_copy(dst_hbm.at[pl.ds(base, CB)], idx_d)
        cs = pltpu.async_copy(hsrc_hbm.at[idx_s], rows_s, sem_s)
        cd = pltpu.async_copy(hdst_hbm.at[idx_d], rows_d, sem_d)
        cs.wait()
        cd.wait()
        pltpu.sync_copy(rows_s, xs_hbm.at[pl.ds(base, CB)])
        pltpu.sync_copy(rows_d, xd_hbm.at[pl.ds(base, CB)])
        return carry

    lax.fori_loop(0, nchunk, body, 0)


def _sc_gather(src_idx, dst_idx, h_src, h_dst):
    n_edges = src_idx.shape[0]
    per_w = n_edges // NW
    nchunk = per_w // CB
    mesh = plsc.VectorSubcoreMesh(core_axis_name="c", subcore_axis_name="s")
    kern = pl.kernel(
        functools.partial(_sc_gather_body, nchunk, per_w),
        mesh=mesh,
        compiler_params=pltpu.CompilerParams(use_tc_tiling_on_sc=False),
        out_type=(
            jax.ShapeDtypeStruct((n_edges, FEAT), jnp.float32),
            jax.ShapeDtypeStruct((n_edges, FEAT), jnp.float32),
        ),
        scratch_types=[
            pltpu.VMEM((CB,), jnp.int32),
            pltpu.VMEM((CB,), jnp.int32),
            pltpu.VMEM((CB, FEAT), jnp.float32),
            pltpu.VMEM((CB, FEAT), jnp.float32),
            pltpu.SemaphoreType.DMA,
            pltpu.SemaphoreType.DMA,
        ],
    )
    return kern(src_idx, dst_idx, h_src, h_dst)


def _tc_body(xs_ref, xd_ref, w1a_ref, w1b_ref, b1p_ref, w2_ref, rw_ref, b2_ref, out_ref):
    xs = xs_ref[...]
    xd = xd_ref[...]
    u = jnp.dot(xs, w1a_ref[...], preferred_element_type=jnp.float32)
    u = u + jnp.dot(xd, w1b_ref[...], preferred_element_type=jnp.float32)
    u = u + b1p_ref[...]
    u = jnp.where(u >= 0, u, 0.2 * u)
    glin = jnp.sum(u * w2_ref[...], axis=1) + b2_ref[0, 0]
    g = jax.nn.sigmoid(glin)
    t = jnp.sum(xs * xd * rw_ref[...], axis=1)
    out_ref[...] = (g * t).reshape(out_ref.shape)


def _tc_mlp(xs, xd, w1a, w1b, b1p, w2row, rwrow, b2s):
    n_edges = xs.shape[0]
    eb = 12800
    lanes = 1600
    nb = n_edges // eb
    out = pl.pallas_call(
        _tc_body,
        grid=(nb,),
        in_specs=[
            pl.BlockSpec((eb, FEAT), lambda e: (e, 0)),
            pl.BlockSpec((eb, FEAT), lambda e: (e, 0)),
            pl.BlockSpec((FEAT, HID), lambda e: (0, 0)),
            pl.BlockSpec((FEAT, HID), lambda e: (0, 0)),
            pl.BlockSpec((1, HID), lambda e: (0, 0)),
            pl.BlockSpec((1, HID), lambda e: (0, 0)),
            pl.BlockSpec((1, FEAT), lambda e: (0, 0)),
            pl.BlockSpec((1, 1), lambda e: (0, 0)),
        ],
        out_specs=pl.BlockSpec((eb // lanes, lanes), lambda e: (e, 0)),
        out_shape=jax.ShapeDtypeStruct((n_edges // lanes, lanes), jnp.float32),
    )(xs, xd, w1a, w1b, b1p, w2row, rwrow, b2s)
    return out.reshape(n_edges)


def kernel(edge_index, h_src, h_dst, rel_weight, W1, b1, W2, b2):
    src_idx = edge_index[0]
    dst_idx = edge_index[1]
    xs, xd = _sc_gather(src_idx, dst_idx, h_src, h_dst)
    w1a = W1[:FEAT]
    w1b = W1[FEAT:2 * FEAT]
    b1p = (rel_weight @ W1[2 * FEAT:] + b1).reshape(1, HID)
    w2row = W2.reshape(1, HID)
    rwrow = rel_weight.reshape(1, FEAT)
    b2s = b2.reshape(1, 1)
    return _tc_mlp(xs, xd, w1a, w1b, b1p, w2row, rwrow, b2s)


# speedup vs baseline: 5.2990x; 1.5371x over previous
import functools

import jax
import jax.numpy as jnp
from jax import lax
from jax.experimental import pallas as pl
from jax.experimental.pallas import tpu as pltpu
from jax.experimental.pallas import tpu_sc as plsc

FEAT = 32
HID = 128

NC = 2
NS = 16
NW = NC * NS

CB = 80


def _sc_gather_body(nchunk, per_w, src_hbm, dst_hbm, hsrc_hbm, hdst_hbm,
                    xs_hbm, xd_hbm, idx_s, idx_d, rows_s, rows_d, sem_s, sem_d):
    wid = lax.axis_index("s") * NC + lax.axis_index("c")

    def body(c, carry):
        base = wid * per_w + c * CB
        pltpu.sync_copy(src_hbm.at[pl.ds(base, CB)], idx_s)
        pltpu.sync_copy(dst_hbm.at[pl.ds(base, CB)], idx_d)
        cs = pltpu.async_copy(hsrc_hbm.at[idx_s], rows_s, sem_s)
        cd = pltpu.async_copy(hdst_hbm.at[idx_d], rows_d, sem_d)
        cs.wait()
        cd.wait()
        pltpu.sync_copy(rows_s, xs_hbm.at[pl.ds(base, CB)])
        pltpu.sync_copy(rows_d, xd_hbm.at[pl.ds(base, CB)])
        return carry

    lax.fori_loop(0, nchunk, body, 0)


def _sc_gather(src_idx, dst_idx, h_src, h_dst):
    n_edges = src_idx.shape[0]
    per_w = n_edges // NW
    nchunk = per_w // CB
    mesh = plsc.VectorSubcoreMesh(core_axis_name="c", subcore_axis_name="s")
    kern = pl.kernel(
        functools.partial(_sc_gather_body, nchunk, per_w),
        mesh=mesh,
        compiler_params=pltpu.CompilerParams(use_tc_tiling_on_sc=False),
        out_type=(
            jax.ShapeDtypeStruct((n_edges, FEAT), jnp.float32),
            jax.ShapeDtypeStruct((n_edges, FEAT), jnp.float32),
        ),
        scratch_types=[
            pltpu.VMEM((CB,), jnp.int32),
            pltpu.VMEM((CB,), jnp.int32),
            pltpu.VMEM((CB, FEAT), jnp.float32),
            pltpu.VMEM((CB, FEAT), jnp.float32),
            pltpu.SemaphoreType.DMA,
            pltpu.SemaphoreType.DMA,
        ],
    )
    return kern(src_idx, dst_idx, h_src, h_dst)


PACK = 4


def _tc_body(xs_ref, xd_ref, bdw1a_ref, bdw1b_ref, b1t_ref, bdw2_ref, bdo_ref,
             relr_ref, b2_ref, out_ref):
    xs = xs_ref[...]
    xd = xd_ref[...]
    u = jnp.dot(xs, bdw1a_ref[...], preferred_element_type=jnp.float32)
    u = u + jnp.dot(xd, bdw1b_ref[...], preferred_element_type=jnp.float32)
    u = u + b1t_ref[...]
    u = jnp.where(u >= 0, u, 0.2 * u)
    glin = jnp.dot(u, bdw2_ref[...], preferred_element_type=jnp.float32) + b2_ref[0, 0]
    g = jax.nn.sigmoid(glin)
    t = jnp.dot(xs * xd * relr_ref[...], bdo_ref[...],
                preferred_element_type=jnp.float32)
    out_ref[...] = g * t


def _tc_mlp(xs_p, xd_p, bdw1a, bdw1b, b1t, bdw2, bdo, relr, b2s):
    n_rows = xs_p.shape[0]
    rb = 3200
    nb = n_rows // rb
    out = pl.pallas_call(
        _tc_body,
        grid=(nb,),
        in_specs=[
            pl.BlockSpec((rb, HID), lambda e: (e, 0)),
            pl.BlockSpec((rb, HID), lambda e: (e, 0)),
            pl.BlockSpec((HID, PACK * HID), lambda e: (0, 0)),
            pl.BlockSpec((HID, PACK * HID), lambda e: (0, 0)),
            pl.BlockSpec((1, PACK * HID), lambda e: (0, 0)),
            pl.BlockSpec((PACK * HID, PACK), lambda e: (0, 0)),
            pl.BlockSpec((HID, PACK), lambda e: (0, 0)),
            pl.BlockSpec((1, HID), lambda e: (0, 0)),
            pl.BlockSpec((1, 1), lambda e: (0, 0)),
        ],
        out_specs=pl.BlockSpec((rb, PACK), lambda e: (e, 0)),
        out_shape=jax.ShapeDtypeStruct((n_rows, PACK), jnp.float32),
    )(xs_p, xd_p, bdw1a, bdw1b, b1t, bdw2, bdo, relr, b2s)
    return out.reshape(n_rows * PACK)


def kernel(edge_index, h_src, h_dst, rel_weight, W1, b1, W2, b2):
    src_idx = edge_index[0]
    dst_idx = edge_index[1]
    n_edges = src_idx.shape[0]
    xs, xd = _sc_gather(src_idx, dst_idx, h_src, h_dst)
    xs_p = xs.reshape(n_edges // PACK, PACK * FEAT)
    xd_p = xd.reshape(n_edges // PACK, PACK * FEAT)
    eye4 = jnp.eye(PACK, dtype=jnp.float32)
    b1p = rel_weight @ W1[2 * FEAT:] + b1
    bdw1a = jnp.kron(eye4, W1[:FEAT])
    bdw1b = jnp.kron(eye4, W1[FEAT:2 * FEAT])
    b1t = jnp.tile(b1p, PACK).reshape(1, PACK * HID)
    bdw2 = jnp.kron(eye4, W2)
    bdo = jnp.kron(eye4, jnp.ones((FEAT, 1), jnp.float32))
    relr = jnp.tile(rel_weight, PACK).reshape(1, HID)
    b2s = b2.reshape(1, 1)
    return _tc_mlp(xs_p, xd_p, bdw1a, bdw1b, b1t, bdw2, bdo, relr, b2s)


# speedup vs baseline: 10.3608x; 1.9552x over previous
import functools

import jax
import jax.numpy as jnp
from jax import lax
from jax.experimental import pallas as pl
from jax.experimental.pallas import tpu as pltpu
from jax.experimental.pallas import tpu_sc as plsc

FEAT = 32
HID = 128

NC = 2
NS = 16
NW = NC * NS

CB = 80


RB = 128
NSLOT = 4


def _sc_gather_body(nrows_tot, rows_per_w, src_hbm, dst_hbm, hsrc_hbm, hdst_hbm,
                    xs_hbm, xd_hbm, idx_s, idx_d, rows_s, rows_d,
                    isem_s, isem_d, gsem, osem):
    wid = lax.axis_index("s") * NC + lax.axis_index("c")
    row0 = wid * rows_per_w
    rend = jnp.minimum(nrows_tot, row0 + rows_per_w)
    ngroups = (rows_per_w + NSLOT - 1) // NSLOT

    def fire(s, p, r):
        pltpu.async_copy(hsrc_hbm.at[idx_s.at[s, p]], rows_s.at[s], gsem[s])
        pltpu.async_copy(hdst_hbm.at[idx_d.at[s, p]], rows_d.at[s], gsem[s])

    def wait_gathers(s):
        pltpu.make_async_copy(hsrc_hbm.at[pl.ds(0, RB)], rows_s.at[s], gsem[s]).wait()
        pltpu.make_async_copy(hdst_hbm.at[pl.ds(0, RB)], rows_d.at[s], gsem[s]).wait()

    def issue_outs(s, r):
        pltpu.async_copy(rows_s.at[s], xs_hbm.at[pl.ds(r * RB, RB)], osem[s])
        pltpu.async_copy(rows_d.at[s], xd_hbm.at[pl.ds(r * RB, RB)], osem[s])

    def wait_outs(s, r):
        pltpu.make_async_copy(rows_s.at[s], xs_hbm.at[pl.ds(r * RB, RB)], osem[s]).wait()
        pltpu.make_async_copy(rows_d.at[s], xd_hbm.at[pl.ds(r * RB, RB)], osem[s]).wait()

    def issue_idx(s, p, r):
        pltpu.async_copy(src_hbm.at[r], idx_s.at[s, p], isem_s[s])
        pltpu.async_copy(dst_hbm.at[r], idx_d.at[s, p], isem_d[s])

    def wait_idx(s, p, r):
        pltpu.make_async_copy(src_hbm.at[r], idx_s.at[s, p], isem_s[s]).wait()
        pltpu.make_async_copy(dst_hbm.at[r], idx_d.at[s, p], isem_d[s]).wait()

    for s in range(NSLOT):
        @pl.when(row0 + s < rend)
        def _(s=s):
            issue_idx(s, 0, row0 + s)

    def body(g, carry):
        p = lax.rem(g, 2)
        for s in range(NSLOT):
            r = row0 + g * NSLOT + s

            @pl.when(jnp.logical_and(g > 0, r - NSLOT < rend))
            def _(s=s, r=r):
                wait_gathers(s)
                issue_outs(s, r - NSLOT)
                wait_outs(s, r - NSLOT)

            @pl.when(r < rend)
            def _(s=s, r=r, p=p):
                wait_idx(s, p, r)
                fire(s, p, r)

            @pl.when(r + NSLOT < rend)
            def _(s=s, r=r, p=p):
                issue_idx(s, 1 - p, r + NSLOT)
        return carry

    lax.fori_loop(0, ngroups, body, 0)

    plast = lax.rem(ngroups - 1, 2)
    del plast
    for s in range(NSLOT):
        r = row0 + (ngroups - 1) * NSLOT + s

        @pl.when(r < rend)
        def _(s=s, r=r):
            wait_gathers(s)
            issue_outs(s, r)
            wait_outs(s, r)


def _sc_gather(src_rows, dst_rows, h_src, h_dst):
    nrows_tot = src_rows.shape[0]
    n_edges = nrows_tot * RB
    rows_per_w = (nrows_tot + NW - 1) // NW
    mesh = plsc.VectorSubcoreMesh(core_axis_name="c", subcore_axis_name="s")
    kern = pl.kernel(
        functools.partial(_sc_gather_body, nrows_tot, rows_per_w),
        mesh=mesh,
        compiler_params=pltpu.CompilerParams(use_tc_tiling_on_sc=False),
        out_type=(
            jax.ShapeDtypeStruct((n_edges, FEAT), jnp.float32),
            jax.ShapeDtypeStruct((n_edges, FEAT), jnp.float32),
        ),
        scratch_types=[
            pltpu.VMEM((NSLOT, 2, RB), jnp.int32),
            pltpu.VMEM((NSLOT, 2, RB), jnp.int32),
            pltpu.VMEM((NSLOT, RB, FEAT), jnp.float32),
            pltpu.VMEM((NSLOT, RB, FEAT), jnp.float32),
            [pltpu.SemaphoreType.DMA] * NSLOT,
            [pltpu.SemaphoreType.DMA] * NSLOT,
            [pltpu.SemaphoreType.DMA] * NSLOT,
            [pltpu.SemaphoreType.DMA] * NSLOT,
        ],
    )
    return kern(src_rows, dst_rows, h_src, h_dst)


PACK = 4


def _tc_body(xs_ref, xd_ref, bdw1a_ref, bdw1b_ref, b1t_ref, bdw2_ref, bdo_ref,
             relr_ref, b2_ref, out_ref):
    xs = xs_ref[...]
    xd = xd_ref[...]
    u = jnp.dot(xs, bdw1a_ref[...], preferred_element_type=jnp.float32)
    u = u + jnp.dot(xd, bdw1b_ref[...], preferred_element_type=jnp.float32)
    u = u + b1t_ref[...]
    u = jnp.where(u >= 0, u, 0.2 * u)
    glin = jnp.dot(u, bdw2_ref[...], preferred_element_type=jnp.float32) + b2_ref[0, 0]
    g = jax.nn.sigmoid(glin)
    t = jnp.dot(xs * xd * relr_ref[...], bdo_ref[...],
                preferred_element_type=jnp.float32)
    out_ref[...] = g * t


def _tc_mlp(xs_p, xd_p, bdw1a, bdw1b, b1t, bdw2, bdo, relr, b2s):
    n_rows = xs_p.shape[0]
    rb = 3200
    nb = n_rows // rb
    out = pl.pallas_call(
        _tc_body,
        grid=(nb,),
        in_specs=[
            pl.BlockSpec((rb, HID), lambda e: (e, 0)),
            pl.BlockSpec((rb, HID), lambda e: (e, 0)),
            pl.BlockSpec((HID, PACK * HID), lambda e: (0, 0)),
            pl.BlockSpec((HID, PACK * HID), lambda e: (0, 0)),
            pl.BlockSpec((1, PACK * HID), lambda e: (0, 0)),
            pl.BlockSpec((PACK * HID, PACK), lambda e: (0, 0)),
            pl.BlockSpec((HID, PACK), lambda e: (0, 0)),
            pl.BlockSpec((1, HID), lambda e: (0, 0)),
            pl.BlockSpec((1, 1), lambda e: (0, 0)),
        ],
        out_specs=pl.BlockSpec((rb, PACK), lambda e: (e, 0)),
        out_shape=jax.ShapeDtypeStruct((n_rows, PACK), jnp.float32),
    )(xs_p, xd_p, bdw1a, bdw1b, b1t, bdw2, bdo, relr, b2s)
    return out.reshape(n_rows * PACK)


def kernel(edge_index, h_src, h_dst, rel_weight, W1, b1, W2, b2):
    n_edges = edge_index.shape[1]
    src_rows = edge_index[0].reshape(n_edges // RB, RB)
    dst_rows = edge_index[1].reshape(n_edges // RB, RB)
    xs, xd = _sc_gather(src_rows, dst_rows, h_src, h_dst)
    xs_p = xs.reshape(n_edges // PACK, PACK * FEAT)
    xd_p = xd.reshape(n_edges // PACK, PACK * FEAT)
    eye4 = jnp.eye(PACK, dtype=jnp.float32)
    b1p = rel_weight @ W1[2 * FEAT:] + b1
    bdw1a = jnp.kron(eye4, W1[:FEAT])
    bdw1b = jnp.kron(eye4, W1[FEAT:2 * FEAT])
    b1t = jnp.tile(b1p, PACK).reshape(1, PACK * HID)
    bdw2 = jnp.kron(eye4, W2)
    bdo = jnp.kron(eye4, jnp.ones((FEAT, 1), jnp.float32))
    relr = jnp.tile(rel_weight, PACK).reshape(1, HID)
    b2s = b2.reshape(1, 1)
    return _tc_mlp(xs_p, xd_p, bdw1a, bdw1b, b1t, bdw2, bdo, relr, b2s)
